# Initial kernel scaffold; baseline (speedup 1.0000x reference)
import jax
import jax.numpy as jnp
from jax.experimental import pallas as pl


def kernel(elec_emb, nuc_up, nuc_down, edge_emb, contr, norm, e_n_i, e_n_k, mask, W_out, b_out, W_edge, W2, b2, scale1, scale2):
    raise NotImplementedError("write your pallas kernel here")


# speedup vs baseline: 3.9392x; 3.9392x over previous
import dataclasses
import functools
import math

import jax
import jax.numpy as jnp
from jax import lax
from jax.experimental import pallas as pl
from jax.experimental.pallas import tpu as pltpu
from jax.experimental.pallas import tpu_sc as plsc

N_ELEC = 10000
N_NUC = 2000
N_EDGE = 320000
D_EDGE = 16
D = 128
GAIN = 1.7868

B_EDGE = 128
NW = 32
BLK_PER_W = 79
E_PAD = NW * BLK_PER_W * B_EDGE
TC_BLK = 2048
ROWS_PER_TILE = 624


def _wmul_body(e_ref, c_ref, w_ref, o_ref):
    o_ref[...] = jnp.dot(e_ref[...], w_ref[...],
                         preferred_element_type=jnp.float32) * c_ref[...]


def _epi_body(x_ref, p_ref, norm_ref, wout_ref, bout_ref, w2_ref, b2_ref, o_ref):
    x = x_ref[...]
    out0 = jnp.dot(x, wout_ref[...], preferred_element_type=jnp.float32) + bout_ref[...]
    seg = p_ref[0] + p_ref[1]
    h = out0 + seg * norm_ref[...]
    h = jax.nn.silu(h) * GAIN
    o = jnp.dot(h, w2_ref[...], preferred_element_type=jnp.float32) + b2_ref[...]
    o = jax.nn.silu(o) * GAIN
    o_ref[...] = (x + o) * jnp.float32(1.0 / math.sqrt(2.0))


def _sc_seg_body(nuc_hbm, eni_hbm, enk_hbm, mask_hbm, wmul_hbm, out_hbm,
                 mask_v, eni_v, kidx_v, rows_v, wmul_v, acc_sh, sem):
    c = lax.axis_index("c")
    s = lax.axis_index("s")
    wid = c * 16 + s

    pltpu.sync_copy(mask_hbm, mask_v)

    zeros16 = jnp.zeros((16,), jnp.float32)

    @pl.loop(0, B_EDGE)
    def _(e):
        for cc in range(8):
            wmul_v[e, pl.ds(cc * 16, 16)] = zeros16

    r0 = s * ROWS_PER_TILE
    for off, n in ((0, 128), (128, 128), (256, 128), (384, 128), (512, 112)):
        pltpu.sync_copy(wmul_v.at[pl.ds(0, n)],
                        acc_sh.at[pl.ds(r0 + off, n)])

    @pl.when(s == 15)
    def _():
        pltpu.sync_copy(wmul_v.at[pl.ds(0, 16)],
                        acc_sh.at[pl.ds(16 * ROWS_PER_TILE, 16)])

    plsc.subcore_barrier()

    @pl.loop(0, BLK_PER_W)
    def _(j):
        base = (wid * BLK_PER_W + j) * B_EDGE
        pltpu.sync_copy(eni_hbm.at[pl.ds(base, B_EDGE)], eni_v)
        pltpu.sync_copy(enk_hbm.at[pl.ds(base, B_EDGE)], kidx_v)

        @pl.loop(0, B_EDGE, step=16)
        def _(q):
            en = eni_v[pl.ds(q, 16)]
            m = plsc.load_gather(mask_v, [en])
            kidx_v[pl.ds(q, 16)] = kidx_v[pl.ds(q, 16)] + (1 - m) * N_NUC

        pltpu.async_copy(nuc_hbm.at[kidx_v], rows_v, sem).wait()
        pltpu.sync_copy(wmul_hbm.at[pl.ds(base, B_EDGE)], wmul_v)

        @pl.loop(0, B_EDGE)
        def _(e):
            for cc in range(8):
                sl = pl.ds(cc * 16, 16)
                rows_v[e, sl] = rows_v[e, sl] * wmul_v[e, sl]

        pltpu.sync_copy(rows_v, acc_sh.at[eni_v], add=True)

    plsc.subcore_barrier()
    pltpu.sync_copy(acc_sh.at[pl.ds(r0, ROWS_PER_TILE)],
                    out_hbm.at[c, pl.ds(r0, ROWS_PER_TILE)])

    @pl.when(s == 15)
    def _():
        pltpu.sync_copy(acc_sh.at[pl.ds(16 * ROWS_PER_TILE, 16)],
                        out_hbm.at[c, pl.ds(16 * ROWS_PER_TILE, 16)])


def kernel(elec_emb, nuc_up, nuc_down, edge_emb, contr, norm, e_n_i, e_n_k,
           mask, W_out, b_out, W_edge, W2, b2, scale1, scale2):
    pad = E_PAD - N_EDGE
    edge_p = jnp.pad(edge_emb, ((0, pad), (0, 0)))
    contr_p = jnp.pad(contr, (0, pad)).reshape(E_PAD, 1)
    eni_p = jnp.pad(e_n_i, (0, pad))
    enk_p = jnp.pad(e_n_k, (0, pad))
    nuc_cat = jnp.concatenate([nuc_up, nuc_down], axis=0)
    mask_i32 = mask.astype(jnp.int32)
    w_edge_s = W_edge * scale1

    wmul = pl.pallas_call(
        _wmul_body,
        grid=(E_PAD // TC_BLK,),
        in_specs=[
            pl.BlockSpec((TC_BLK, D_EDGE), lambda i: (i, 0)),
            pl.BlockSpec((TC_BLK, 1), lambda i: (i, 0)),
            pl.BlockSpec((D_EDGE, D), lambda i: (0, 0)),
        ],
        out_specs=pl.BlockSpec((TC_BLK, D), lambda i: (i, 0)),
        out_shape=jax.ShapeDtypeStruct((E_PAD, D), jnp.float32),
    )(edge_p, contr_p, w_edge_s)

    sc_params = pltpu.CompilerParams()
    if "needs_layout_passes" in pltpu.CompilerParams.__dataclass_fields__:
        sc_params = dataclasses.replace(sc_params, needs_layout_passes=False)
    sc_call = pl.kernel(
        _sc_seg_body,
        compiler_params=sc_params,
        out_type=jax.ShapeDtypeStruct((2, N_ELEC, D), jnp.float32),
        mesh=plsc.VectorSubcoreMesh(core_axis_name="c", subcore_axis_name="s"),
        scratch_types=[
            pltpu.VMEM((N_ELEC,), jnp.int32),
            pltpu.VMEM((B_EDGE,), jnp.int32),
            pltpu.VMEM((B_EDGE,), jnp.int32),
            pltpu.VMEM((B_EDGE, D), jnp.float32),
            pltpu.VMEM((B_EDGE, D), jnp.float32),
            pltpu.VMEM_SHARED((N_ELEC, D), jnp.float32),
            pltpu.SemaphoreType.DMA,
        ],
    )
    partials = sc_call(nuc_cat, eni_p, enk_p, mask_i32, wmul)

    out = pl.pallas_call(
        _epi_body,
        grid=(10,),
        in_specs=[
            pl.BlockSpec((1000, D), lambda i: (i, 0)),
            pl.BlockSpec((2, 1000, D), lambda i: (0, i, 0)),
            pl.BlockSpec((1000, 1), lambda i: (i, 0)),
            pl.BlockSpec((D, D), lambda i: (0, 0)),
            pl.BlockSpec((1, D), lambda i: (0, 0)),
            pl.BlockSpec((D, D), lambda i: (0, 0)),
            pl.BlockSpec((1, D), lambda i: (0, 0)),
        ],
        out_specs=pl.BlockSpec((1000, D), lambda i: (i, 0)),
        out_shape=jax.ShapeDtypeStruct((N_ELEC, D), jnp.float32),
    )(elec_emb, partials, (norm * scale2).reshape(N_ELEC, 1),
      W_out * scale2, (b_out * scale2).reshape(1, D), W2, b2.reshape(1, D))
    return out


# speedup vs baseline: 4.7395x; 1.2032x over previous
import dataclasses
import functools
import math

import jax
import jax.numpy as jnp
from jax import lax
from jax.experimental import pallas as pl
from jax.experimental.pallas import tpu as pltpu
from jax.experimental.pallas import tpu_sc as plsc

N_ELEC = 10000
N_NUC = 2000
N_EDGE = 320000
D_EDGE = 16
D = 128
GAIN = 1.7868

B_EDGE = 80
NW = 32
BLK_PER_W = 125
TC_BLK = 2000
TC_GRID = N_EDGE // TC_BLK
ROWS_PER_TILE = 624


def _wmul_body(e_ref, c_ref, wlo_ref, whi_ref, o_ref):
    e = e_ref[...]
    c = c_ref[...]
    lo = jnp.dot(e, wlo_ref[...], preferred_element_type=jnp.float32) * c
    hi = jnp.dot(e, whi_ref[...], preferred_element_type=jnp.float32) * c
    ulo = (jax.lax.bitcast_convert_type(lo, jnp.int32) + 0x8000) >> 16
    uhi = jax.lax.bitcast_convert_type(hi, jnp.int32) + 0x8000
    o_ref[...] = (uhi & jnp.int32(-65536)) | (ulo & jnp.int32(0xFFFF))


def _epi_body(x_ref, p_ref, norm_ref, wout_ref, bout_ref, w2_ref, b2_ref, o_ref):
    x = x_ref[...]
    out0 = jnp.dot(x, wout_ref[...], preferred_element_type=jnp.float32) + bout_ref[...]
    seg = p_ref[0] + p_ref[1]
    h = out0 + seg * norm_ref[...]
    h = jax.nn.silu(h) * GAIN
    o = jnp.dot(h, w2_ref[...], preferred_element_type=jnp.float32) + b2_ref[...]
    o = jax.nn.silu(o) * GAIN
    o_ref[...] = (x + o) * jnp.float32(1.0 / math.sqrt(2.0))


def _sc_seg_body(nuc_hbm, eni_hbm, enk_hbm, mask_hbm, wmul_hbm, out_hbm,
                 mval_v, eni0, eni1, kidx0, kidx1, rows0, rows1, wm0, wm1,
                 acc_sh, sg0, sg1, sw0, sw1, ss0, ss1, sm):
    c = lax.axis_index("c")
    s = lax.axis_index("s")
    wid = c * 16 + s
    eni = (eni0, eni1)
    kidx = (kidx0, kidx1)
    rows = (rows0, rows1)
    wm = (wm0, wm1)
    sg = (sg0, sg1)
    sw = (sw0, sw1)
    ss = (ss0, ss1)

    zeros16 = jnp.zeros((16,), jnp.float32)

    @pl.loop(0, B_EDGE)
    def _(e):
        for cc in range(8):
            rows0[e, pl.ds(cc * 16, 16)] = zeros16

    r0 = s * ROWS_PER_TILE
    for off, n in ((0, 80), (80, 80), (160, 80), (240, 80), (320, 80),
                   (400, 80), (480, 80), (560, 64)):
        pltpu.sync_copy(rows0.at[pl.ds(0, n)], acc_sh.at[pl.ds(r0 + off, n)])

    @pl.when(s == 15)
    def _():
        pltpu.sync_copy(rows0.at[pl.ds(0, 16)],
                        acc_sh.at[pl.ds(16 * ROWS_PER_TILE, 16)])

    plsc.subcore_barrier()

    def fetch(j, b):
        base = (wid * BLK_PER_W + j) * B_EDGE
        pltpu.sync_copy(eni_hbm.at[pl.ds(base, B_EDGE)], eni[b])
        pltpu.sync_copy(enk_hbm.at[pl.ds(base, B_EDGE)], kidx[b])
        pltpu.async_copy(mask_hbm.at[eni[b]], mval_v, sm).wait()

        @pl.loop(0, B_EDGE, step=16)
        def _(q):
            m = mval_v[pl.ds(q, 16)]
            kidx[b][pl.ds(q, 16)] = kidx[b][pl.ds(q, 16)] + (1 - m) * N_NUC

        pltpu.async_copy(nuc_hbm.at[kidx[b]], rows[b], sg[b])
        pltpu.async_copy(wmul_hbm.at[pl.ds(base, B_EDGE)], wm[b], sw[b])

    def process(b):
        pltpu.make_async_copy(nuc_hbm.at[kidx[b]], rows[b], sg[b]).wait()
        pltpu.make_async_copy(wmul_hbm.at[pl.ds(0, B_EDGE)], wm[b], sw[b]).wait()

        @pl.loop(0, B_EDGE)
        def _(e):
            for q in range(4):
                w = wm[b][e, pl.ds(q * 16, 16)]
                flo = plsc.bitcast(w << 16, jnp.float32)
                fhi = plsc.bitcast(w & jnp.int32(-65536), jnp.float32)
                sl_lo = pl.ds(q * 32, 16)
                sl_hi = pl.ds(q * 32 + 16, 16)
                rows[b][e, sl_lo] = rows[b][e, sl_lo] * flo
                rows[b][e, sl_hi] = rows[b][e, sl_hi] * fhi

        pltpu.async_copy(rows[b], acc_sh.at[eni[b]], ss[b], add=True)

    def wait_scatter(b):
        pltpu.make_async_copy(rows[b], acc_sh.at[eni[b]], ss[b]).wait()

    fetch(0, 0)
    fetch(1, 1)

    @pl.loop(0, BLK_PER_W // 2)
    def _(t):
        j0 = 2 * t
        process(0)
        process(1)
        wait_scatter(0)
        fetch(j0 + 2, 0)

        @pl.when(t < BLK_PER_W // 2 - 1)
        def _():
            wait_scatter(1)
            fetch(j0 + 3, 1)

    process(0)
    wait_scatter(0)
    wait_scatter(1)

    plsc.subcore_barrier()
    pltpu.sync_copy(acc_sh.at[pl.ds(r0, ROWS_PER_TILE)],
                    out_hbm.at[c, pl.ds(r0, ROWS_PER_TILE)])

    @pl.when(s == 15)
    def _():
        pltpu.sync_copy(acc_sh.at[pl.ds(16 * ROWS_PER_TILE, 16)],
                        out_hbm.at[c, pl.ds(16 * ROWS_PER_TILE, 16)])


def kernel(elec_emb, nuc_up, nuc_down, edge_emb, contr, norm, e_n_i, e_n_k,
           mask, W_out, b_out, W_edge, W2, b2, scale1, scale2):
    contr_2d = contr.reshape(N_EDGE, 1)
    nuc_cat = jnp.concatenate([nuc_up, nuc_down], axis=0)
    mask_i32 = mask.astype(jnp.int32)
    w_edge_s = W_edge * scale1
    lo_map = jnp.arange(64) // 16 * 32 + jnp.arange(64) % 16
    w_lo = w_edge_s[:, lo_map]
    w_hi = w_edge_s[:, lo_map + 16]

    wmul = pl.pallas_call(
        _wmul_body,
        grid=(TC_GRID,),
        in_specs=[
            pl.BlockSpec((TC_BLK, D_EDGE), lambda i: (i, 0)),
            pl.BlockSpec((TC_BLK, 1), lambda i: (i, 0)),
            pl.BlockSpec((D_EDGE, 64), lambda i: (0, 0)),
            pl.BlockSpec((D_EDGE, 64), lambda i: (0, 0)),
        ],
        out_specs=pl.BlockSpec((TC_BLK, 64), lambda i: (i, 0)),
        out_shape=jax.ShapeDtypeStruct((N_EDGE, 64), jnp.int32),
    )(edge_emb, contr_2d, w_lo, w_hi)

    sc_params = pltpu.CompilerParams()
    if "needs_layout_passes" in pltpu.CompilerParams.__dataclass_fields__:
        sc_params = dataclasses.replace(sc_params, needs_layout_passes=False)
    sc_call = pl.kernel(
        _sc_seg_body,
        compiler_params=sc_params,
        out_type=jax.ShapeDtypeStruct((2, N_ELEC, D), jnp.float32),
        mesh=plsc.VectorSubcoreMesh(core_axis_name="c", subcore_axis_name="s"),
        scratch_types=[
            pltpu.VMEM((B_EDGE,), jnp.int32),
            pltpu.VMEM((B_EDGE,), jnp.int32),
            pltpu.VMEM((B_EDGE,), jnp.int32),
            pltpu.VMEM((B_EDGE,), jnp.int32),
            pltpu.VMEM((B_EDGE,), jnp.int32),
            pltpu.VMEM((B_EDGE, D), jnp.float32),
            pltpu.VMEM((B_EDGE, D), jnp.float32),
            pltpu.VMEM((B_EDGE, 64), jnp.int32),
            pltpu.VMEM((B_EDGE, 64), jnp.int32),
            pltpu.VMEM_SHARED((N_ELEC, D), jnp.float32),
            pltpu.SemaphoreType.DMA,
            pltpu.SemaphoreType.DMA,
            pltpu.SemaphoreType.DMA,
            pltpu.SemaphoreType.DMA,
            pltpu.SemaphoreType.DMA,
            pltpu.SemaphoreType.DMA,
            pltpu.SemaphoreType.DMA,
        ],
    )
    partials = sc_call(nuc_cat, e_n_i, e_n_k, mask_i32, wmul)

    out = pl.pallas_call(
        _epi_body,
        grid=(10,),
        in_specs=[
            pl.BlockSpec((1000, D), lambda i: (i, 0)),
            pl.BlockSpec((2, 1000, D), lambda i: (0, i, 0)),
            pl.BlockSpec((1000, 1), lambda i: (i, 0)),
            pl.BlockSpec((D, D), lambda i: (0, 0)),
            pl.BlockSpec((1, D), lambda i: (0, 0)),
            pl.BlockSpec((D, D), lambda i: (0, 0)),
            pl.BlockSpec((1, D), lambda i: (0, 0)),
        ],
        out_specs=pl.BlockSpec((1000, D), lambda i: (i, 0)),
        out_shape=jax.ShapeDtypeStruct((N_ELEC, D), jnp.float32),
    )(elec_emb, partials, (norm * scale2).reshape(N_ELEC, 1),
      W_out * scale2, (b_out * scale2).reshape(1, D), W2, b2.reshape(1, D))
    return out


# speedup vs baseline: 7.0340x; 1.4841x over previous
import dataclasses
import functools
import math

import jax
import jax.numpy as jnp
from jax import lax
from jax.experimental import pallas as pl
from jax.experimental.pallas import tpu as pltpu
from jax.experimental.pallas import tpu_sc as plsc

N_ELEC = 10000
N_NUC = 2000
N_EDGE = 320000
D_EDGE = 16
D = 128
GAIN = 1.7868

B_EDGE = 80
NW = 32
BLK_PER_W = 125
TC_BLK = 2000
TC_GRID = N_EDGE // TC_BLK
ROWS_PER_TILE = 624


def _wmul_body(e_ref, wlo_ref, whi_ref, o_ref):
    e = e_ref[...]
    lo = jnp.dot(e, wlo_ref[...], preferred_element_type=jnp.float32)
    hi = jnp.dot(e, whi_ref[...], preferred_element_type=jnp.float32)
    ulo = (jax.lax.bitcast_convert_type(lo, jnp.int32) + 0x8000) >> 16
    uhi = jax.lax.bitcast_convert_type(hi, jnp.int32) + 0x8000
    o_ref[...] = (uhi & jnp.int32(-65536)) | (ulo & jnp.int32(0xFFFF))


def _epi_body(x_ref, p_ref, norm_ref, wout_ref, bout_ref, w2_ref, b2_ref, o_ref):
    x = x_ref[...]
    out0 = jnp.dot(x, wout_ref[...], preferred_element_type=jnp.float32) + bout_ref[...]
    seg = p_ref[0] + p_ref[1]
    h = out0 + seg * norm_ref[...]
    h = jax.nn.silu(h) * GAIN
    o = jnp.dot(h, w2_ref[...], preferred_element_type=jnp.float32) + b2_ref[...]
    o = jax.nn.silu(o) * GAIN
    o_ref[...] = (x + o) * jnp.float32(1.0 / math.sqrt(2.0))


def _sc_seg_body(nuc_hbm, eni_hbm, enk_hbm, mask_hbm, wmul_hbm, out_hbm,
                 eni_sb, enk_sb, mval, eni0, eni1, kidx0, kidx1,
                 rows0, rows1, wm0, wm1, acc_sh,
                 sg0, sg1, sw0, sw1, ss0, ss1, sm0, sm1, sm2, sm3):
    c = lax.axis_index("c")
    s = lax.axis_index("s")
    wid = c * 16 + s
    eni = (eni0, eni1)
    kidx = (kidx0, kidx1)
    rows = (rows0, rows1)
    wm = (wm0, wm1)
    sg = (sg0, sg1)
    sw = (sw0, sw1)
    ss = (ss0, ss1)
    sm = (sm0, sm1, sm2, sm3)

    zeros16 = jnp.zeros((16,), jnp.float32)

    @pl.loop(0, B_EDGE)
    def _(e):
        for cc in range(8):
            rows0[e, pl.ds(cc * 16, 16)] = zeros16

    r0 = s * ROWS_PER_TILE
    for off, n in ((0, 80), (80, 80), (160, 80), (240, 80), (320, 80),
                   (400, 80), (480, 80), (560, 64)):
        pltpu.sync_copy(rows0.at[pl.ds(0, n)], acc_sh.at[pl.ds(r0 + off, n)])

    @pl.when(s == 15)
    def _():
        pltpu.sync_copy(rows0.at[pl.ds(0, 16)],
                        acc_sh.at[pl.ds(16 * ROWS_PER_TILE, 16)])

    plsc.subcore_barrier()

    def sb_setup(t):
        sb_base = (wid * BLK_PER_W + 4 * t) * B_EDGE
        pltpu.sync_copy(eni_hbm.at[pl.ds(sb_base, 4 * B_EDGE)], eni_sb)
        pltpu.sync_copy(enk_hbm.at[pl.ds(sb_base, 4 * B_EDGE)], enk_sb)
        for p in range(4):
            pltpu.async_copy(mask_hbm.at[eni_sb.at[pl.ds(p * B_EDGE, B_EDGE)]],
                             mval.at[p], sm[p])

    def ff(j, b, p):
        pltpu.make_async_copy(mask_hbm.at[eni[b]], mval.at[p], sm[p]).wait()

        @pl.loop(0, B_EDGE, step=16)
        def _(q):
            env = eni_sb[pl.ds(p * B_EDGE + q, 16)]
            m = mval[p, pl.ds(q, 16)]
            kidx[b][pl.ds(q, 16)] = (enk_sb[pl.ds(p * B_EDGE + q, 16)]
                                     + (1 - m) * N_NUC)
            eni[b][pl.ds(q, 16)] = env

        base = (wid * BLK_PER_W + j) * B_EDGE
        pltpu.async_copy(nuc_hbm.at[kidx[b]], rows[b], sg[b])
        pltpu.async_copy(wmul_hbm.at[pl.ds(base, B_EDGE)], wm[b], sw[b])

    def process(b):
        pltpu.make_async_copy(nuc_hbm.at[kidx[b]], rows[b], sg[b]).wait()
        pltpu.make_async_copy(wmul_hbm.at[pl.ds(0, B_EDGE)], wm[b], sw[b]).wait()

        @pl.loop(0, B_EDGE)
        def _(e):
            for q in range(4):
                w = wm[b][e, pl.ds(q * 16, 16)]
                flo = plsc.bitcast(w << 16, jnp.float32)
                fhi = plsc.bitcast(w & jnp.int32(-65536), jnp.float32)
                sl_lo = pl.ds(q * 32, 16)
                sl_hi = pl.ds(q * 32 + 16, 16)
                rows[b][e, sl_lo] = rows[b][e, sl_lo] * flo
                rows[b][e, sl_hi] = rows[b][e, sl_hi] * fhi

        pltpu.async_copy(rows[b], acc_sh.at[eni[b]], ss[b], add=True)

    def wait_scatter(b):
        pltpu.make_async_copy(rows[b], acc_sh.at[eni[b]], ss[b]).wait()

    sb_setup(0)
    ff(0, 0, 0)
    ff(1, 1, 1)

    @pl.loop(0, 31)
    def _(t):
        j0 = 4 * t
        process(0)
        process(1)
        wait_scatter(0)
        ff(j0 + 2, 0, 2)
        wait_scatter(1)
        ff(j0 + 3, 1, 3)

        @pl.when(t < 30)
        def _():
            sb_setup(t + 1)

        process(0)
        process(1)

        @pl.when(t < 30)
        def _():
            wait_scatter(0)
            ff(j0 + 4, 0, 0)
            wait_scatter(1)
            ff(j0 + 5, 1, 1)

    wait_scatter(0)
    base_t = (wid * BLK_PER_W + BLK_PER_W - 1) * B_EDGE
    pltpu.sync_copy(eni_hbm.at[pl.ds(base_t, B_EDGE)], eni[0])
    pltpu.sync_copy(enk_hbm.at[pl.ds(base_t, B_EDGE)], kidx[0])
    pltpu.async_copy(mask_hbm.at[eni[0]], mval.at[0], sm[0]).wait()

    @pl.loop(0, B_EDGE, step=16)
    def _(q):
        m = mval[0, pl.ds(q, 16)]
        kidx[0][pl.ds(q, 16)] = kidx[0][pl.ds(q, 16)] + (1 - m) * N_NUC

    pltpu.async_copy(nuc_hbm.at[kidx[0]], rows[0], sg[0])
    pltpu.async_copy(wmul_hbm.at[pl.ds(base_t, B_EDGE)], wm[0], sw[0])
    process(0)
    wait_scatter(0)
    wait_scatter(1)

    plsc.subcore_barrier()
    pltpu.sync_copy(acc_sh.at[pl.ds(r0, ROWS_PER_TILE)],
                    out_hbm.at[c, pl.ds(r0, ROWS_PER_TILE)])

    @pl.when(s == 15)
    def _():
        pltpu.sync_copy(acc_sh.at[pl.ds(16 * ROWS_PER_TILE, 16)],
                        out_hbm.at[c, pl.ds(16 * ROWS_PER_TILE, 16)])


def kernel(elec_emb, nuc_up, nuc_down, edge_emb, contr, norm, e_n_i, e_n_k,
           mask, W_out, b_out, W_edge, W2, b2, scale1, scale2):
    nuc_cat = jnp.concatenate([nuc_up, nuc_down], axis=0)
    mask_i32 = mask.astype(jnp.int32)
    w_edge_s = W_edge * scale1
    j64 = jnp.arange(64)
    lo_map = j64 // 16 * 32 + j64 % 16
    w_lo = w_edge_s[:, lo_map].astype(jnp.bfloat16)
    w_hi = w_edge_s[:, lo_map + 16].astype(jnp.bfloat16)
    edge_bf = (edge_emb * contr[:, None]).astype(jnp.bfloat16)

    wmul = pl.pallas_call(
        _wmul_body,
        grid=(TC_GRID,),
        in_specs=[
            pl.BlockSpec((TC_BLK, D_EDGE), lambda i: (i, 0)),
            pl.BlockSpec((D_EDGE, 64), lambda i: (0, 0)),
            pl.BlockSpec((D_EDGE, 64), lambda i: (0, 0)),
        ],
        out_specs=pl.BlockSpec((TC_BLK, 64), lambda i: (i, 0)),
        out_shape=jax.ShapeDtypeStruct((N_EDGE, 64), jnp.int32),
    )(edge_bf, w_lo, w_hi)

    sc_params = pltpu.CompilerParams()
    if "needs_layout_passes" in pltpu.CompilerParams.__dataclass_fields__:
        sc_params = dataclasses.replace(sc_params, needs_layout_passes=False)
    sc_call = pl.kernel(
        _sc_seg_body,
        compiler_params=sc_params,
        out_type=jax.ShapeDtypeStruct((2, N_ELEC, D), jnp.float32),
        mesh=plsc.VectorSubcoreMesh(core_axis_name="c", subcore_axis_name="s"),
        scratch_types=[
            pltpu.VMEM((4 * B_EDGE,), jnp.int32),
            pltpu.VMEM((4 * B_EDGE,), jnp.int32),
            pltpu.VMEM((4, B_EDGE), jnp.int32),
            pltpu.VMEM((B_EDGE,), jnp.int32),
            pltpu.VMEM((B_EDGE,), jnp.int32),
            pltpu.VMEM((B_EDGE,), jnp.int32),
            pltpu.VMEM((B_EDGE,), jnp.int32),
            pltpu.VMEM((B_EDGE, D), jnp.float32),
            pltpu.VMEM((B_EDGE, D), jnp.float32),
            pltpu.VMEM((B_EDGE, 64), jnp.int32),
            pltpu.VMEM((B_EDGE, 64), jnp.int32),
            pltpu.VMEM_SHARED((N_ELEC, D), jnp.float32),
        ] + [pltpu.SemaphoreType.DMA] * 10,
    )
    partials = sc_call(nuc_cat, e_n_i, e_n_k, mask_i32, wmul)

    out = pl.pallas_call(
        _epi_body,
        grid=(10,),
        in_specs=[
            pl.BlockSpec((1000, D), lambda i: (i, 0)),
            pl.BlockSpec((2, 1000, D), lambda i: (0, i, 0)),
            pl.BlockSpec((1000, 1), lambda i: (i, 0)),
            pl.BlockSpec((D, D), lambda i: (0, 0)),
            pl.BlockSpec((1, D), lambda i: (0, 0)),
            pl.BlockSpec((D, D), lambda i: (0, 0)),
            pl.BlockSpec((1, D), lambda i: (0, 0)),
        ],
        out_specs=pl.BlockSpec((1000, D), lambda i: (i, 0)),
        out_shape=jax.ShapeDtypeStruct((N_ELEC, D), jnp.float32),
    )(elec_emb, partials, (norm * scale2).reshape(N_ELEC, 1),
      W_out * scale2, (b_out * scale2).reshape(1, D), W2, b2.reshape(1, D))
    return out


# speedup vs baseline: 8.2435x; 1.1719x over previous
import dataclasses
import functools
import math

import jax
import jax.numpy as jnp
from jax import lax
from jax.experimental import pallas as pl
from jax.experimental.pallas import tpu as pltpu
from jax.experimental.pallas import tpu_sc as plsc

N_ELEC = 10000
N_NUC = 2000
N_EDGE = 320000
D_EDGE = 16
D = 128
GAIN = 1.7868

B_EDGE = 80
NW = 32
BLK_PER_W = 125
TC_BLK = 8000
TC_GRID = N_EDGE // TC_BLK
ROWS_PER_TILE = 624


def _wmul_body(e_ref, wlo_ref, whi_ref, o_ref):
    e = e_ref[...]
    lo = jnp.dot(e, wlo_ref[...], preferred_element_type=jnp.float32)
    hi = jnp.dot(e, whi_ref[...], preferred_element_type=jnp.float32)
    ulo = (jax.lax.bitcast_convert_type(lo, jnp.int32) + 0x8000) >> 16
    uhi = jax.lax.bitcast_convert_type(hi, jnp.int32) + 0x8000
    o_ref[...] = (uhi & jnp.int32(-65536)) | (ulo & jnp.int32(0xFFFF))


def _epi_body(x_ref, p_ref, norm_ref, wout_ref, bout_ref, w2_ref, b2_ref, o_ref):
    x = x_ref[...]
    out0 = jnp.dot(x, wout_ref[...], preferred_element_type=jnp.float32) + bout_ref[...]
    seg = p_ref[0] + p_ref[1]
    h = out0 + seg * norm_ref[...]
    h = jax.nn.silu(h) * GAIN
    o = jnp.dot(h, w2_ref[...], preferred_element_type=jnp.float32) + b2_ref[...]
    o = jax.nn.silu(o) * GAIN
    o_ref[...] = (x + o) * jnp.float32(1.0 / math.sqrt(2.0))


def _sc_seg_body(nuc_hbm, eni_hbm, enk_hbm, mask_hbm, wmul_hbm, out_hbm,
                 eni_sb, enk_sb, mval, eni0, eni1, kidx0, kidx1,
                 rows0, rows1, wm0, wm1, acc_sh,
                 sg0, sg1, sw0, sw1, ss0, ss1, sm0, sm1, sm2, sm3):
    c = lax.axis_index("c")
    s = lax.axis_index("s")
    wid = c * 16 + s
    eni = (eni0, eni1)
    kidx = (kidx0, kidx1)
    rows = (rows0, rows1)
    wm = (wm0, wm1)
    sg = (sg0, sg1)
    sw = (sw0, sw1)
    ss = (ss0, ss1)
    sm = (sm0, sm1, sm2, sm3)

    zeros16 = jnp.zeros((16,), jnp.float32)

    @pl.loop(0, B_EDGE)
    def _(e):
        for cc in range(8):
            rows0[e, pl.ds(cc * 16, 16)] = zeros16

    r0 = s * ROWS_PER_TILE
    for off, n in ((0, 80), (80, 80), (160, 80), (240, 80), (320, 80),
                   (400, 80), (480, 80), (560, 64)):
        pltpu.sync_copy(rows0.at[pl.ds(0, n)], acc_sh.at[pl.ds(r0 + off, n)])

    @pl.when(s == 15)
    def _():
        pltpu.sync_copy(rows0.at[pl.ds(0, 16)],
                        acc_sh.at[pl.ds(16 * ROWS_PER_TILE, 16)])

    plsc.subcore_barrier()

    def sb_setup(t):
        sb_base = (wid * BLK_PER_W + 4 * t) * B_EDGE
        pltpu.sync_copy(eni_hbm.at[pl.ds(sb_base, 4 * B_EDGE)], eni_sb)
        pltpu.sync_copy(enk_hbm.at[pl.ds(sb_base, 4 * B_EDGE)], enk_sb)
        for p in range(4):
            pltpu.async_copy(mask_hbm.at[eni_sb.at[pl.ds(p * B_EDGE, B_EDGE)]],
                             mval.at[p], sm[p])

    def ff(j, b, p):
        pltpu.make_async_copy(mask_hbm.at[eni[b]], mval.at[p], sm[p]).wait()

        @pl.loop(0, B_EDGE, step=16)
        def _(q):
            env = eni_sb[pl.ds(p * B_EDGE + q, 16)]
            m = mval[p, pl.ds(q, 16)]
            kidx[b][pl.ds(q, 16)] = (enk_sb[pl.ds(p * B_EDGE + q, 16)]
                                     + (1 - m) * N_NUC)
            eni[b][pl.ds(q, 16)] = env

        base = (wid * BLK_PER_W + j) * B_EDGE
        pltpu.async_copy(nuc_hbm.at[kidx[b]], rows[b], sg[b])
        pltpu.async_copy(wmul_hbm.at[pl.ds(base, B_EDGE)], wm[b], sw[b])

    def process(b):
        pltpu.make_async_copy(nuc_hbm.at[kidx[b]], rows[b], sg[b]).wait()
        pltpu.make_async_copy(wmul_hbm.at[pl.ds(0, B_EDGE)], wm[b], sw[b]).wait()

        @plsc.parallel_loop(0, B_EDGE)
        def _(e):
            for q in range(4):
                w = wm[b][e, pl.ds(q * 16, 16)]
                flo = plsc.bitcast(w << 16, jnp.float32)
                fhi = plsc.bitcast(w & jnp.int32(-65536), jnp.float32)
                sl_lo = pl.ds(q * 32, 16)
                sl_hi = pl.ds(q * 32 + 16, 16)
                rows[b][e, sl_lo] = rows[b][e, sl_lo] * flo
                rows[b][e, sl_hi] = rows[b][e, sl_hi] * fhi

        pltpu.async_copy(rows[b], acc_sh.at[eni[b]], ss[b], add=True)

    def wait_scatter(b):
        pltpu.make_async_copy(rows[b], acc_sh.at[eni[b]], ss[b]).wait()

    sb_setup(0)
    ff(0, 0, 0)
    ff(1, 1, 1)

    @pl.loop(0, 31)
    def _(t):
        j0 = 4 * t
        process(0)
        process(1)
        wait_scatter(0)
        ff(j0 + 2, 0, 2)
        wait_scatter(1)
        ff(j0 + 3, 1, 3)

        @pl.when(t < 30)
        def _():
            sb_setup(t + 1)

        process(0)
        process(1)

        @pl.when(t < 30)
        def _():
            wait_scatter(0)
            ff(j0 + 4, 0, 0)
            wait_scatter(1)
            ff(j0 + 5, 1, 1)

    wait_scatter(0)
    base_t = (wid * BLK_PER_W + BLK_PER_W - 1) * B_EDGE
    pltpu.sync_copy(eni_hbm.at[pl.ds(base_t, B_EDGE)], eni[0])
    pltpu.sync_copy(enk_hbm.at[pl.ds(base_t, B_EDGE)], kidx[0])
    pltpu.async_copy(mask_hbm.at[eni[0]], mval.at[0], sm[0]).wait()

    @pl.loop(0, B_EDGE, step=16)
    def _(q):
        m = mval[0, pl.ds(q, 16)]
        kidx[0][pl.ds(q, 16)] = kidx[0][pl.ds(q, 16)] + (1 - m) * N_NUC

    pltpu.async_copy(nuc_hbm.at[kidx[0]], rows[0], sg[0])
    pltpu.async_copy(wmul_hbm.at[pl.ds(base_t, B_EDGE)], wm[0], sw[0])
    process(0)
    wait_scatter(0)
    wait_scatter(1)

    plsc.subcore_barrier()
    pltpu.sync_copy(acc_sh.at[pl.ds(r0, ROWS_PER_TILE)],
                    out_hbm.at[c, pl.ds(r0, ROWS_PER_TILE)])

    @pl.when(s == 15)
    def _():
        pltpu.sync_copy(acc_sh.at[pl.ds(16 * ROWS_PER_TILE, 16)],
                        out_hbm.at[c, pl.ds(16 * ROWS_PER_TILE, 16)])


def kernel(elec_emb, nuc_up, nuc_down, edge_emb, contr, norm, e_n_i, e_n_k,
           mask, W_out, b_out, W_edge, W2, b2, scale1, scale2):
    nuc_cat = jnp.concatenate([nuc_up, nuc_down], axis=0)
    mask_i32 = mask.astype(jnp.int32)
    w_edge_s = W_edge * scale1
    j64 = jnp.arange(64)
    lo_map = j64 // 16 * 32 + j64 % 16
    w_lo = w_edge_s[:, lo_map].astype(jnp.bfloat16)
    w_hi = w_edge_s[:, lo_map + 16].astype(jnp.bfloat16)
    edge_bf = (edge_emb * contr[:, None]).astype(jnp.bfloat16)

    wmul = pl.pallas_call(
        _wmul_body,
        grid=(TC_GRID,),
        in_specs=[
            pl.BlockSpec((TC_BLK, D_EDGE), lambda i: (i, 0)),
            pl.BlockSpec((D_EDGE, 64), lambda i: (0, 0)),
            pl.BlockSpec((D_EDGE, 64), lambda i: (0, 0)),
        ],
        out_specs=pl.BlockSpec((TC_BLK, 64), lambda i: (i, 0)),
        out_shape=jax.ShapeDtypeStruct((N_EDGE, 64), jnp.int32),
    )(edge_bf, w_lo, w_hi)

    sc_params = pltpu.CompilerParams()
    if "needs_layout_passes" in pltpu.CompilerParams.__dataclass_fields__:
        sc_params = dataclasses.replace(sc_params, needs_layout_passes=False)
    sc_call = pl.kernel(
        _sc_seg_body,
        compiler_params=sc_params,
        out_type=jax.ShapeDtypeStruct((2, N_ELEC, D), jnp.float32),
        mesh=plsc.VectorSubcoreMesh(core_axis_name="c", subcore_axis_name="s"),
        scratch_types=[
            pltpu.VMEM((4 * B_EDGE,), jnp.int32),
            pltpu.VMEM((4 * B_EDGE,), jnp.int32),
            pltpu.VMEM((4, B_EDGE), jnp.int32),
            pltpu.VMEM((B_EDGE,), jnp.int32),
            pltpu.VMEM((B_EDGE,), jnp.int32),
            pltpu.VMEM((B_EDGE,), jnp.int32),
            pltpu.VMEM((B_EDGE,), jnp.int32),
            pltpu.VMEM((B_EDGE, D), jnp.float32),
            pltpu.VMEM((B_EDGE, D), jnp.float32),
            pltpu.VMEM((B_EDGE, 64), jnp.int32),
            pltpu.VMEM((B_EDGE, 64), jnp.int32),
            pltpu.VMEM_SHARED((N_ELEC, D), jnp.float32),
        ] + [pltpu.SemaphoreType.DMA] * 10,
    )
    partials = sc_call(nuc_cat, e_n_i, e_n_k, mask_i32, wmul)

    out = pl.pallas_call(
        _epi_body,
        grid=(10,),
        in_specs=[
            pl.BlockSpec((1000, D), lambda i: (i, 0)),
            pl.BlockSpec((2, 1000, D), lambda i: (0, i, 0)),
            pl.BlockSpec((1000, 1), lambda i: (i, 0)),
            pl.BlockSpec((D, D), lambda i: (0, 0)),
            pl.BlockSpec((1, D), lambda i: (0, 0)),
            pl.BlockSpec((D, D), lambda i: (0, 0)),
            pl.BlockSpec((1, D), lambda i: (0, 0)),
        ],
        out_specs=pl.BlockSpec((1000, D), lambda i: (i, 0)),
        out_shape=jax.ShapeDtypeStruct((N_ELEC, D), jnp.float32),
    )(elec_emb, partials, (norm * scale2).reshape(N_ELEC, 1),
      W_out * scale2, (b_out * scale2).reshape(1, D), W2, b2.reshape(1, D))
    return out
